# baseline (speedup 1.0000x reference)
import functools

import jax
import jax.numpy as jnp
from jax.experimental import pallas as pl


def _support_kernel(x_ref, w_ref, out_ref):
    w = w_ref[...]
    r, i, j, k = jnp.split(w, 4, axis=1)
    r2 = jnp.concatenate([r, -i, -j, -k], axis=0)
    i2 = jnp.concatenate([i, r, -k, j], axis=0)
    j2 = jnp.concatenate([j, k, r, -i], axis=0)
    k2 = jnp.concatenate([k, -j, i, r], axis=0)
    hamilton = jnp.concatenate([r2, i2, j2, k2], axis=1)
    out_ref[...] = jnp.dot(x_ref[...], hamilton,
                           preferred_element_type=jnp.float32)


def _matmul_kernel(adj_ref, sup_ref, out_ref, s1_ref, s2_ref):
    i = pl.program_id(0)
    o = jnp.dot(adj_ref[...], sup_ref[...],
                preferred_element_type=jnp.float32)

    @pl.when(i == 0)
    def _():
        s1_ref[...] = jnp.zeros_like(s1_ref)
        s2_ref[...] = jnp.zeros_like(s2_ref)

    out_ref[...] = o
    s1_ref[...] += jnp.sum(o, axis=0, keepdims=True)
    s2_ref[...] += jnp.sum(o * o, axis=0, keepdims=True)


def _bn_kernel(out_ref, s1_ref, s2_ref, g_ref, b_ref, y_ref, *, n):
    mean = s1_ref[...] / n
    var = s2_ref[...] / n - mean * mean
    inv = jax.lax.rsqrt(var + 1e-5)
    y = (out_ref[...] - mean) * (inv * g_ref[...]) + b_ref[...]
    y_ref[...] = jnp.tanh(y)


def kernel(x, adj, weight, gamma, beta):
    n, fin = x.shape
    fout = weight.shape[1]

    support = pl.pallas_call(
        _support_kernel,
        out_shape=jax.ShapeDtypeStruct((n, fout), jnp.float32),
    )(x, weight)

    rb = 400
    nr = n // rb
    out, s1, s2 = pl.pallas_call(
        _matmul_kernel,
        grid=(nr,),
        in_specs=[
            pl.BlockSpec((rb, n), lambda i: (i, 0)),
            pl.BlockSpec((n, fout), lambda i: (0, 0)),
        ],
        out_specs=[
            pl.BlockSpec((rb, fout), lambda i: (i, 0)),
            pl.BlockSpec((1, fout), lambda i: (0, 0)),
            pl.BlockSpec((1, fout), lambda i: (0, 0)),
        ],
        out_shape=[
            jax.ShapeDtypeStruct((n, fout), jnp.float32),
            jax.ShapeDtypeStruct((1, fout), jnp.float32),
            jax.ShapeDtypeStruct((1, fout), jnp.float32),
        ],
    )(adj, support)

    g2 = gamma.reshape(1, fout)
    b2 = beta.reshape(1, fout)
    y = pl.pallas_call(
        functools.partial(_bn_kernel, n=float(n)),
        grid=(nr,),
        in_specs=[
            pl.BlockSpec((rb, fout), lambda i: (i, 0)),
            pl.BlockSpec((1, fout), lambda i: (0, 0)),
            pl.BlockSpec((1, fout), lambda i: (0, 0)),
            pl.BlockSpec((1, fout), lambda i: (0, 0)),
            pl.BlockSpec((1, fout), lambda i: (0, 0)),
        ],
        out_specs=pl.BlockSpec((rb, fout), lambda i: (i, 0)),
        out_shape=jax.ShapeDtypeStruct((n, fout), jnp.float32),
    )(out, s1, s2, g2, b2)
    return y


# speedup vs baseline: 1.1594x; 1.1594x over previous
import jax
import jax.numpy as jnp
from jax.experimental import pallas as pl
from jax.experimental.pallas import tpu as pltpu


def _make_hamilton(w):
    r, i, j, k = jnp.split(w, 4, axis=1)
    r2 = jnp.concatenate([r, -i, -j, -k], axis=0)
    i2 = jnp.concatenate([i, r, -k, j], axis=0)
    j2 = jnp.concatenate([j, k, r, -i], axis=0)
    k2 = jnp.concatenate([k, -j, i, r], axis=0)
    return jnp.concatenate([r2, i2, j2, k2], axis=1)


def _fused_kernel(x_ref, adj_ref, w_ref, g_ref, b_ref, y_ref,
                  sup_ref, s1_ref, s2_ref, *, rb, nr, n):
    i = pl.program_id(0)

    @pl.when(i == 0)
    def _():
        hamilton = _make_hamilton(w_ref[...])
        sup_ref[...] = jnp.dot(x_ref[...], hamilton,
                               preferred_element_type=jnp.float32)
        s1_ref[...] = jnp.zeros_like(s1_ref)
        s2_ref[...] = jnp.zeros_like(s2_ref)

    o = jnp.dot(adj_ref[...], sup_ref[...], preferred_element_type=jnp.float32)
    y_ref[pl.ds(i * rb, rb), :] = o
    s1_ref[...] += jnp.sum(o, axis=0, keepdims=True)
    s2_ref[...] += jnp.sum(o * o, axis=0, keepdims=True)

    @pl.when(i == nr - 1)
    def _():
        mean = s1_ref[...] / n
        var = s2_ref[...] / n - mean * mean
        inv = jax.lax.rsqrt(var + 1e-5)
        scale = inv * g_ref[...]
        shift = b_ref[...] - mean * scale
        y_ref[...] = jnp.tanh(y_ref[...] * scale + shift)


def kernel(x, adj, weight, gamma, beta):
    n, _ = x.shape
    fout = weight.shape[1]
    rb = 200
    nr = n // rb

    import functools
    y = pl.pallas_call(
        functools.partial(_fused_kernel, rb=rb, nr=nr, n=float(n)),
        grid=(nr,),
        in_specs=[
            pl.BlockSpec((n, x.shape[1]), lambda i: (0, 0)),
            pl.BlockSpec((rb, n), lambda i: (i, 0)),
            pl.BlockSpec(weight.shape, lambda i: (0, 0)),
            pl.BlockSpec((1, fout), lambda i: (0, 0)),
            pl.BlockSpec((1, fout), lambda i: (0, 0)),
        ],
        out_specs=pl.BlockSpec((n, fout), lambda i: (0, 0)),
        out_shape=jax.ShapeDtypeStruct((n, fout), jnp.float32),
        scratch_shapes=[
            pltpu.VMEM((n, fout), jnp.float32),
            pltpu.VMEM((1, fout), jnp.float32),
            pltpu.VMEM((1, fout), jnp.float32),
        ],
    )(x, adj, weight, gamma.reshape(1, fout), beta.reshape(1, fout))
    return y


# speedup vs baseline: 1.1615x; 1.0018x over previous
import jax
import jax.numpy as jnp
from jax.experimental import pallas as pl
from jax.experimental.pallas import tpu as pltpu


def _make_hamilton(w):
    r, i, j, k = jnp.split(w, 4, axis=1)
    r2 = jnp.concatenate([r, -i, -j, -k], axis=0)
    i2 = jnp.concatenate([i, r, -k, j], axis=0)
    j2 = jnp.concatenate([j, k, r, -i], axis=0)
    k2 = jnp.concatenate([k, -j, i, r], axis=0)
    return jnp.concatenate([r2, i2, j2, k2], axis=1)


def _fused_kernel(x_ref, adj_ref, w_ref, g_ref, b_ref, y_ref,
                  sup_ref, s1_ref, s2_ref, *, rb, nr, n):
    i = pl.program_id(0)

    @pl.when(i == 0)
    def _():
        hamilton = _make_hamilton(w_ref[...])
        sup_ref[...] = jnp.dot(x_ref[...], hamilton,
                               preferred_element_type=jnp.float32)
        s1_ref[...] = jnp.zeros_like(s1_ref)
        s2_ref[...] = jnp.zeros_like(s2_ref)

    o = jnp.dot(adj_ref[...], sup_ref[...], preferred_element_type=jnp.float32)
    y_ref[pl.ds(i * rb, rb), :] = o
    s1_ref[...] += jnp.sum(o, axis=0, keepdims=True)
    s2_ref[...] += jnp.sum(o * o, axis=0, keepdims=True)

    @pl.when(i == nr - 1)
    def _():
        mean = s1_ref[...] / n
        var = s2_ref[...] / n - mean * mean
        inv = jax.lax.rsqrt(var + 1e-5)
        scale = inv * g_ref[...]
        shift = b_ref[...] - mean * scale
        y_ref[...] = jnp.tanh(y_ref[...] * scale + shift)


def kernel(x, adj, weight, gamma, beta):
    n, _ = x.shape
    fout = weight.shape[1]
    rb = 400
    nr = n // rb

    import functools
    y = pl.pallas_call(
        functools.partial(_fused_kernel, rb=rb, nr=nr, n=float(n)),
        grid=(nr,),
        in_specs=[
            pl.BlockSpec((n, x.shape[1]), lambda i: (0, 0)),
            pl.BlockSpec((rb, n), lambda i: (i, 0)),
            pl.BlockSpec(weight.shape, lambda i: (0, 0)),
            pl.BlockSpec((1, fout), lambda i: (0, 0)),
            pl.BlockSpec((1, fout), lambda i: (0, 0)),
        ],
        out_specs=pl.BlockSpec((n, fout), lambda i: (0, 0)),
        out_shape=jax.ShapeDtypeStruct((n, fout), jnp.float32),
        scratch_shapes=[
            pltpu.VMEM((n, fout), jnp.float32),
            pltpu.VMEM((1, fout), jnp.float32),
            pltpu.VMEM((1, fout), jnp.float32),
        ],
    )(x, adj, weight, gamma.reshape(1, fout), beta.reshape(1, fout))
    return y
